# baseline (speedup 1.0000x reference)
import jax
import jax.numpy as jnp
from jax.experimental import pallas as pl
from jax.experimental.pallas import tpu as pltpu

_N = 18432
_K = 1024
_D = 64
_BLOCK = 3072
_COMMIT = 0.25


def _vq_block(z_ref, emb_ref, embm2_ref, z2_ref, e2_ref, iotaf_ref,
              idx_ref, q_ref, part_ref):
    z = z_ref[...]
    mm2 = jax.lax.dot_general(z, embm2_ref[...], (((1,), (1,)), ((), ())),
                              preferred_element_type=jnp.float32)
    d = (z2_ref[...] + e2_ref[...]) + mm2
    m = jnp.min(d, axis=1, keepdims=True)
    iotaf = iotaf_ref[...]
    cand = jnp.where(d == m, iotaf, jnp.float32(_K))
    idxf = jnp.min(cand, axis=1, keepdims=True)
    idx_ref[...] = idxf.astype(jnp.int32)
    onehot = (iotaf == idxf).astype(jnp.float32)
    q = jax.lax.dot_general(onehot, emb_ref[...], (((1,), (0,)), ((), ())),
                            preferred_element_type=jnp.float32)
    q_ref[...] = q
    part_ref[...] = jnp.sum(m)[None, None, None]


def kernel(z_flat, emb):
    z2 = jnp.sum(z_flat ** 2, axis=1, keepdims=True)
    e2 = jnp.sum(emb ** 2, axis=1)[None, :]
    nblocks = _N // _BLOCK
    idx, q, part = pl.pallas_call(
        _vq_block,
        grid=(nblocks,),
        in_specs=[
            pl.BlockSpec((_BLOCK, _D), lambda i: (i, 0)),
            pl.BlockSpec((_K, _D), lambda i: (0, 0)),
            pl.BlockSpec((_K, _D), lambda i: (0, 0)),
            pl.BlockSpec((_BLOCK, 1), lambda i: (i, 0)),
            pl.BlockSpec((1, _K), lambda i: (0, 0)),
            pl.BlockSpec((1, _K), lambda i: (0, 0)),
        ],
        out_specs=[
            pl.BlockSpec((_BLOCK, 1), lambda i: (i, 0)),
            pl.BlockSpec((_BLOCK, _D), lambda i: (i, 0)),
            pl.BlockSpec((1, 1, 1), lambda i: (i, 0, 0)),
        ],
        out_shape=[
            jax.ShapeDtypeStruct((_N, 1), jnp.int32),
            jax.ShapeDtypeStruct((_N, _D), jnp.float32),
            jax.ShapeDtypeStruct((nblocks, 1, 1), jnp.float32),
        ],
        compiler_params=pltpu.CompilerParams(
            dimension_semantics=("parallel",)),
    )(z_flat, emb, -2.0 * emb, z2, e2,
      jnp.arange(_K, dtype=jnp.float32)[None, :])
    loss = jnp.sum(part) * ((1.0 + _COMMIT) / (_N * _D))
    return (loss, q, idx[:, 0])


# speedup vs baseline: 1.0319x; 1.0319x over previous
import jax
import jax.numpy as jnp
from jax.experimental import pallas as pl
from jax.experimental.pallas import tpu as pltpu

_N = 18432
_K = 1024
_D = 64
_BLOCK = 3072
_COMMIT = 0.25


def _vq_block(z_ref, emb_ref, embm2_ref, z2_ref, e2_ref, iotaf_ref,
              idx_ref, q_ref, part_ref):
    z = z_ref[...]
    mm2 = jax.lax.dot_general(z, embm2_ref[...], (((1,), (1,)), ((), ())),
                              preferred_element_type=jnp.float32)
    d = (z2_ref[...] + e2_ref[...]) + mm2
    m = jnp.min(d, axis=1, keepdims=True)
    iotaf = iotaf_ref[...]
    cand = jnp.where(d == m, iotaf, jnp.float32(_K))
    idxf = jnp.min(cand, axis=1, keepdims=True)
    idx_ref[...] = idxf[:, 0].astype(jnp.int32)
    onehot = (iotaf == idxf).astype(jnp.float32)
    q = jax.lax.dot_general(onehot, emb_ref[...], (((1,), (0,)), ((), ())),
                            preferred_element_type=jnp.float32)
    q_ref[...] = q
    part_ref[...] = jnp.sum(m)[None, None, None]


def kernel(z_flat, emb):
    z2 = jnp.sum(z_flat ** 2, axis=1, keepdims=True)
    e2 = jnp.sum(emb ** 2, axis=1)[None, :]
    nblocks = _N // _BLOCK
    idx, q, part = pl.pallas_call(
        _vq_block,
        grid=(nblocks,),
        in_specs=[
            pl.BlockSpec((_BLOCK, _D), lambda i: (i, 0)),
            pl.BlockSpec((_K, _D), lambda i: (0, 0)),
            pl.BlockSpec((_K, _D), lambda i: (0, 0)),
            pl.BlockSpec((_BLOCK, 1), lambda i: (i, 0)),
            pl.BlockSpec((1, _K), lambda i: (0, 0)),
            pl.BlockSpec((1, _K), lambda i: (0, 0)),
        ],
        out_specs=[
            pl.BlockSpec((_BLOCK,), lambda i: (i,)),
            pl.BlockSpec((_BLOCK, _D), lambda i: (i, 0)),
            pl.BlockSpec((1, 1, 1), lambda i: (i, 0, 0)),
        ],
        out_shape=[
            jax.ShapeDtypeStruct((_N,), jnp.int32),
            jax.ShapeDtypeStruct((_N, _D), jnp.float32),
            jax.ShapeDtypeStruct((nblocks, 1, 1), jnp.float32),
        ],
        compiler_params=pltpu.CompilerParams(
            dimension_semantics=("parallel",)),
    )(z_flat, emb, -2.0 * emb, z2, e2,
      jnp.arange(_K, dtype=jnp.float32)[None, :])
    loss = jnp.sum(part) * ((1.0 + _COMMIT) / (_N * _D))
    return (loss, q, idx)
